# Initial kernel scaffold; baseline (speedup 1.0000x reference)
import jax
import jax.numpy as jnp
from jax.experimental import pallas as pl


def kernel(x_v, x_f, wv_weight, wv_u, wv_c, wv_bias, wf_weight, wf_u, wf_c, wf_bias, edge_index_v, edge_index_f):
    raise NotImplementedError("write your pallas kernel here")


# speedup vs baseline: 20.1987x; 20.1987x over previous
import functools

import jax
import jax.numpy as jnp
from jax import lax
from jax.experimental import pallas as pl
from jax.experimental.pallas import tpu as pltpu
from jax.experimental.pallas import tpu_sc as plsc

N = 10000
E = 320000
D = 128
H = 4
C = 128
HC = H * C

NC = 2
NS = 16
NP = 10240
ET = E // NS
NCH = ET // 16
NCHP = NCH + 2
HALF = NCH // 2
SROWS = NP // NS
DROWS = NP // 128

BN = 320


def _pre_body(x_ref, w_ref, u_ref, c_ref, h_ref, p_ref, lmx_ref):
    x = x_ref[0]
    w = w_ref[0]
    u = u_ref[0]
    cc = c_ref[0]
    h = jnp.dot(x, w, preferred_element_type=jnp.float32)
    h_ref[...] = h
    p_ref[...] = jnp.dot(x, u, preferred_element_type=jnp.float32)
    e = jnp.exp(cc - jnp.max(cc))
    qc = e / jnp.sum(e)
    lm = h[:, 0:C] * qc[0:1, 0:1]
    for hh in range(1, H):
        lm = lm + h[:, hh * C:(hh + 1) * C] * qc[0:1, hh:hh + 1]
    lmx_ref[...] = lm


def _pre(x_all, w_all, u_all, c_all):
    nb = NP // BN
    return pl.pallas_call(
        _pre_body,
        grid=(2, nb),
        in_specs=[
            pl.BlockSpec((1, BN, D), lambda c, i: (c, i, 0)),
            pl.BlockSpec((1, D, HC), lambda c, i: (c, 0, 0)),
            pl.BlockSpec((1, D, H), lambda c, i: (c, 0, 0)),
            pl.BlockSpec((1, 1, H), lambda c, i: (c, 0, 0)),
        ],
        out_specs=[
            pl.BlockSpec((BN, HC), lambda c, i: (c * nb + i, 0)),
            pl.BlockSpec((BN, H), lambda c, i: (c * nb + i, 0)),
            pl.BlockSpec((BN, C), lambda c, i: (c * nb + i, 0)),
        ],
        out_shape=[
            jax.ShapeDtypeStruct((2 * NP, HC), jnp.float32),
            jax.ShapeDtypeStruct((2 * NP, H), jnp.float32),
            jax.ShapeDtypeStruct((2 * NP, C), jnp.float32),
        ],
    )(x_all, w_all, u_all, c_all)


def _lane_bcast(v, k):
    idx = jnp.full((16, 1), k, jnp.int32)
    dn = lax.GatherDimensionNumbers(offset_dims=(),
                                    collapsed_slice_dims=(0,),
                                    start_index_map=(0,))
    return lax.gather(v, idx, dn, (1,),
                      mode=lax.GatherScatterMode.PROMISE_IN_BOUNDS)


def _sc_body(h_hbm, p_hbm, lmx_hbm, epack_hbm, bias_hbm, out_hbm,
             epack, rows0, rows1, psb0, psb1, pdb0, pdb1, msg, msgd,
             acc, degbuf, bias_vm, sr0, sr1, sp0, sp1, sq0, sq1):
    cid = lax.axis_index("c")
    sid = lax.axis_index("s")
    rows = (rows0, rows1)
    psb = (psb0, psb1)
    pdb = (pdb0, pdb1)
    sr = (sr0, sr1)
    sp = (sp0, sp1)
    sq = (sq0, sq1)
    iota = lax.iota(jnp.int32, 16)
    zerov = jnp.zeros((16,), jnp.float32)
    nbase = cid * NP
    pbase = cid * NP * H

    pltpu.sync_copy(epack_hbm.at[pl.ds(cid * E + sid * ET, ET // 2)],
                    epack)
    pltpu.sync_copy(bias_hbm.at[cid], bias_vm)
    for k in range(16):
        for j in range(8):
            msgd[k, pl.ds(j * 16, 16)] = zerov
    rbase = sid * SROWS
    pltpu.sync_copy(lmx_hbm.at[pl.ds(cid * NP + rbase, SROWS)],
                    acc.at[pl.ds(rbase, SROWS)])
    pltpu.sync_copy(msgd.at[pl.ds(0, DROWS // NS)],
                    acc.at[pl.ds(NP + sid * (DROWS // NS), DROWS // NS)])

    cvec = bias_vm[pl.ds(128, 16)]
    cs = [_lane_bcast(cvec, hh) for hh in range(H)]

    def issue(ch, b):
        chc = jnp.minimum(ch, NCH - 1)
        loff = jnp.where(chc >= HALF, chc - HALF, chc) * 16
        epk = epack[pl.ds(loff, 16)]
        s16 = epk & 16383
        d16 = lax.shift_right_logical(epk, 14)
        pltpu.async_copy(h_hbm.at[s16 + nbase], rows[b], sr[b])
        s4 = s16 * H + pbase
        d4 = d16 * H + pbase
        for hh in range(H):
            pltpu.async_copy(p_hbm.at[s4 + hh],
                             psb[b].at[pl.ds(hh * 16, 16)], sp[b])
            pltpu.async_copy(p_hbm.at[d4 + hh],
                             pdb[b].at[pl.ds(hh * 16, 16)], sq[b])
        return s16, d16

    sv0, dv0 = issue(jnp.int32(0), 0)
    sv1, dv1 = issue(jnp.int32(1), 1)
    plsc.subcore_barrier()

    def body(i, carry):
        prev_didx, sv, dv = carry
        sv = list(sv)
        dv = list(dv)
        for b in range(2):
            ch = i * 2 + b
            pltpu.make_async_copy(h_hbm.at[iota], rows[b],
                                  sr[b]).wait()
            for hh in range(H):
                pltpu.make_async_copy(p_hbm.at[iota],
                                      psb[b].at[pl.ds(hh * 16, 16)],
                                      sp[b]).wait()
                pltpu.make_async_copy(p_hbm.at[iota],
                                      pdb[b].at[pl.ds(hh * 16, 16)],
                                      sq[b]).wait()
            s16 = sv[b]
            d16 = dv[b]
            lg = [psb[b][pl.ds(hh * 16, 16)] - pdb[b][pl.ds(hh * 16, 16)]
                  + cs[hh] for hh in range(H)]
            m = jnp.maximum(jnp.maximum(lg[0], lg[1]),
                            jnp.maximum(lg[2], lg[3]))
            ex = [jnp.exp(v - m) for v in lg]
            ssum = (ex[0] + ex[1]) + (ex[2] + ex[3])
            live = (s16 != d16) & (ch < NCH)
            maskf = jnp.where(live, 1.0, 0.0).astype(jnp.float32)
            r = maskf / ssum
            qvecs = [ex[hh] * r for hh in range(H)]
            for k in range(16):
                qs = [_lane_bcast(qvecs[hh], k) for hh in range(H)]
                accv = [rows[b][k, pl.ds(j * 16, 16)] * qs[0]
                        for j in range(8)]
                for hh in range(1, H):
                    for j in range(8):
                        accv[j] = accv[j] + (
                            rows[b][k, pl.ds(hh * C + j * 16, 16)] * qs[hh])
                for j in range(8):
                    msg[k, pl.ds(j * 16, 16)] = accv[j]
            @pl.when(ch == HALF - 2)
            def _():
                pltpu.sync_copy(
                    epack_hbm.at[pl.ds(cid * E + sid * ET + ET // 2,
                                       ET // 2)], epack)
            sv[b], dv[b] = issue(ch + 2, b)
            dlane = d16 & 127
            plsc.store_scatter(msgd, [iota, prev_didx], zerov)
            plsc.store_scatter(msgd, [iota, dlane], maskf)
            prev_didx = dlane
            pltpu.sync_copy(msg, acc.at[d16], add=True)
            pltpu.sync_copy(msgd, acc.at[NP + lax.shift_right_logical(
                d16, 7)], add=True)
        return prev_didx, tuple(sv), tuple(dv)

    lax.fori_loop(0, NCHP // 2, body, (iota, (sv0, sv1), (dv0, dv1)))
    for b in range(2):
        pltpu.make_async_copy(h_hbm.at[iota], rows[b], sr[b]).wait()
        for hh in range(H):
            pltpu.make_async_copy(p_hbm.at[iota],
                                  psb[b].at[pl.ds(hh * 16, 16)],
                                  sp[b]).wait()
            pltpu.make_async_copy(p_hbm.at[iota],
                                  pdb[b].at[pl.ds(hh * 16, 16)],
                                  sq[b]).wait()
    plsc.subcore_barrier()

    pltpu.sync_copy(acc.at[pl.ds(NP + sid * (DROWS // NS), DROWS // NS)],
                    degbuf)

    def post(it, carry):
        roff = rbase + it * 16
        pltpu.sync_copy(acc.at[pl.ds(roff, 16)], msg)
        dv = degbuf[lax.shift_right_logical(it, 3), pl.ds((it & 7) * 16, 16)]
        inv = 1.0 / (dv + 1.0)
        for rr in range(16):
            ir = _lane_bcast(inv, rr)
            for j in range(8):
                v = msg[rr, pl.ds(j * 16, 16)] * ir + \
                    bias_vm[pl.ds(j * 16, 16)]
                msgd[rr, pl.ds(j * 16, 16)] = jnp.where(v >= 0.0, v,
                                                        v * 0.2)
        pltpu.sync_copy(msgd, out_hbm.at[cid, pl.ds(roff, 16)])
        return carry

    lax.fori_loop(0, SROWS // 16, post, jnp.int32(0))


@functools.lru_cache(maxsize=1)
def _sc_feast():
    return functools.partial(
        pl.kernel,
        out_type=jax.ShapeDtypeStruct((2, NP, 128), jnp.float32),
        mesh=plsc.VectorSubcoreMesh(core_axis_name="c",
                                    subcore_axis_name="s"),
        compiler_params=pltpu.CompilerParams(needs_layout_passes=False),
        scratch_types=[
            pltpu.VMEM((ET // 2,), jnp.int32),
            pltpu.VMEM((16, HC), jnp.float32),
            pltpu.VMEM((16, HC), jnp.float32),
            pltpu.VMEM((64,), jnp.float32),
            pltpu.VMEM((64,), jnp.float32),
            pltpu.VMEM((64,), jnp.float32),
            pltpu.VMEM((64,), jnp.float32),
            pltpu.VMEM((16, C), jnp.float32),
            pltpu.VMEM((16, C), jnp.float32),
            pltpu.VMEM_SHARED((NP + DROWS, C), jnp.float32),
            pltpu.VMEM((DROWS // NS, C), jnp.float32),
            pltpu.VMEM((144,), jnp.float32),
            pltpu.SemaphoreType.DMA,
            pltpu.SemaphoreType.DMA,
            pltpu.SemaphoreType.DMA,
            pltpu.SemaphoreType.DMA,
            pltpu.SemaphoreType.DMA,
            pltpu.SemaphoreType.DMA,
        ],
    )(_sc_body)


def kernel(x_v, x_f, wv_weight, wv_u, wv_c, wv_bias,
           wf_weight, wf_u, wf_c, wf_bias, edge_index_v, edge_index_f):
    x_all = jnp.stack([x_v, x_f])
    x_all = jnp.pad(x_all, ((0, 0), (0, NP - N), (0, 0)))
    w_all = jnp.stack([wv_weight, wf_weight])
    u_all = jnp.stack([wv_u, wf_u])
    c_all = jnp.stack([wv_c, wf_c]).reshape(2, 1, H)
    bias_all = jnp.concatenate(
        [jnp.stack([wv_bias, wf_bias]),
         jnp.stack([wv_c, wf_c]),
         jnp.zeros((2, 12), jnp.float32)], axis=1)
    src_all = jnp.concatenate([edge_index_v[0], edge_index_f[0]])
    dst_all = jnp.concatenate([edge_index_v[1], edge_index_f[1]])
    epack_all = src_all | (dst_all << 14)
    h_all, p_all, lmx_all = _pre(x_all, w_all, u_all, c_all)
    p_flat = p_all.reshape(2 * NP * H)
    out = _sc_feast()(h_all, p_flat, lmx_all, epack_all, bias_all)
    return out[0, :N], out[1, :N]


# speedup vs baseline: 21.5168x; 1.0653x over previous
import functools

import jax
import jax.numpy as jnp
from jax import lax
from jax.experimental import pallas as pl
from jax.experimental.pallas import tpu as pltpu
from jax.experimental.pallas import tpu_sc as plsc

N = 10000
E = 320000
D = 128
H = 4
C = 128
HC = H * C

NC = 2
NS = 16
NP = 10240
ET = E // NS
NCH = ET // 16
NCHP = NCH + 2
HALF = NCH // 2
SROWS = NP // NS
DROWS = NP // 128

BN = 320


def _pre_body(x_ref, w_ref, u_ref, c_ref, h_ref, p_ref, lmx_ref):
    x = x_ref[0]
    w = w_ref[0]
    u = u_ref[0]
    cc = c_ref[0]
    h = jnp.dot(x, w, preferred_element_type=jnp.float32)
    h_ref[...] = h
    p_ref[...] = jnp.dot(x, u, preferred_element_type=jnp.float32)
    e = jnp.exp(cc - jnp.max(cc))
    qc = e / jnp.sum(e)
    lm = h[:, 0:C] * qc[0:1, 0:1]
    for hh in range(1, H):
        lm = lm + h[:, hh * C:(hh + 1) * C] * qc[0:1, hh:hh + 1]
    lmx_ref[...] = lm


def _pre(x_all, w_all, u_all, c_all):
    nb = NP // BN
    return pl.pallas_call(
        _pre_body,
        grid=(2, nb),
        in_specs=[
            pl.BlockSpec((1, BN, D), lambda c, i: (c, i, 0)),
            pl.BlockSpec((1, D, HC), lambda c, i: (c, 0, 0)),
            pl.BlockSpec((1, D, H), lambda c, i: (c, 0, 0)),
            pl.BlockSpec((1, 1, H), lambda c, i: (c, 0, 0)),
        ],
        out_specs=[
            pl.BlockSpec((BN, HC), lambda c, i: (c * nb + i, 0)),
            pl.BlockSpec((BN, H), lambda c, i: (c * nb + i, 0)),
            pl.BlockSpec((BN, C), lambda c, i: (c * nb + i, 0)),
        ],
        out_shape=[
            jax.ShapeDtypeStruct((2 * NP, HC), jnp.float32),
            jax.ShapeDtypeStruct((2 * NP, H), jnp.float32),
            jax.ShapeDtypeStruct((2 * NP, C), jnp.float32),
        ],
    )(x_all, w_all, u_all, c_all)


def _lane_bcast(v, k):
    idx = jnp.full((16, 1), k, jnp.int32)
    dn = lax.GatherDimensionNumbers(offset_dims=(),
                                    collapsed_slice_dims=(0,),
                                    start_index_map=(0,))
    return lax.gather(v, idx, dn, (1,),
                      mode=lax.GatherScatterMode.PROMISE_IN_BOUNDS)


def _sc_body(h_hbm, p_hbm, lmx_hbm, epack_hbm, bias_hbm, out_hbm,
             epack, rows0, rows1, psb0, psb1, pdb0, pdb1, msg0, msg1,
             msgd0, msgd1, acc, degbuf, bias_vm, sr0, sr1, sp0, sp1,
             sq0, sq1, sm0, sm1):
    cid = lax.axis_index("c")
    sid = lax.axis_index("s")
    rows = (rows0, rows1)
    psb = (psb0, psb1)
    pdb = (pdb0, pdb1)
    sr = (sr0, sr1)
    sp = (sp0, sp1)
    sq = (sq0, sq1)
    sm = (sm0, sm1)
    msg = (msg0, msg1)
    msgd = (msgd0, msgd1)
    iota = lax.iota(jnp.int32, 16)
    zerov = jnp.zeros((16,), jnp.float32)
    nbase = cid * NP
    pbase = cid * NP * H

    pltpu.sync_copy(epack_hbm.at[pl.ds(cid * E + sid * ET, ET // 2)],
                    epack)
    pltpu.sync_copy(bias_hbm.at[cid], bias_vm)
    for bb in range(2):
        for k in range(16):
            for j in range(8):
                msgd[bb][k, pl.ds(j * 16, 16)] = zerov
    rbase = sid * SROWS
    pltpu.sync_copy(lmx_hbm.at[pl.ds(cid * NP + rbase, SROWS)],
                    acc.at[pl.ds(rbase, SROWS)])
    pltpu.sync_copy(msgd0.at[pl.ds(0, DROWS // NS)],
                    acc.at[pl.ds(NP + sid * (DROWS // NS), DROWS // NS)])

    cvec = bias_vm[pl.ds(128, 16)]
    cs = [_lane_bcast(cvec, hh) for hh in range(H)]

    def issue(ch, b):
        chc = jnp.minimum(ch, NCH - 1)
        loff = jnp.where(chc >= HALF, chc - HALF, chc) * 16
        epk = epack[pl.ds(loff, 16)]
        s16 = epk & 16383
        d16 = lax.shift_right_logical(epk, 14)
        pltpu.async_copy(h_hbm.at[s16 + nbase], rows[b], sr[b])
        s4 = s16 * H + pbase
        d4 = d16 * H + pbase
        for hh in range(H):
            pltpu.async_copy(p_hbm.at[s4 + hh],
                             psb[b].at[pl.ds(hh * 16, 16)], sp[b])
            pltpu.async_copy(p_hbm.at[d4 + hh],
                             pdb[b].at[pl.ds(hh * 16, 16)], sq[b])
        return s16, d16

    sv0, dv0 = issue(jnp.int32(0), 0)
    sv1, dv1 = issue(jnp.int32(1), 1)
    plsc.subcore_barrier()
    for b in range(2):
        pltpu.async_copy(msgd[b], acc.at[iota], sm[b], add=True)
        pltpu.async_copy(msgd[b], acc.at[iota], sm[b], add=True)

    def body(i, carry):
        prev_didx, sv, dv = carry
        prev_didx = list(prev_didx)
        sv = list(sv)
        dv = list(dv)
        for b in range(2):
            ch = i * 2 + b
            pltpu.make_async_copy(h_hbm.at[iota], rows[b],
                                  sr[b]).wait()
            for hh in range(H):
                pltpu.make_async_copy(p_hbm.at[iota],
                                      psb[b].at[pl.ds(hh * 16, 16)],
                                      sp[b]).wait()
                pltpu.make_async_copy(p_hbm.at[iota],
                                      pdb[b].at[pl.ds(hh * 16, 16)],
                                      sq[b]).wait()
            s16 = sv[b]
            d16 = dv[b]
            pltpu.make_async_copy(msg[b], acc.at[iota], sm[b]).wait()
            pltpu.make_async_copy(msgd[b], acc.at[iota], sm[b]).wait()
            lg = [psb[b][pl.ds(hh * 16, 16)] - pdb[b][pl.ds(hh * 16, 16)]
                  + cs[hh] for hh in range(H)]
            m = jnp.maximum(jnp.maximum(lg[0], lg[1]),
                            jnp.maximum(lg[2], lg[3]))
            ex = [jnp.exp(v - m) for v in lg]
            ssum = (ex[0] + ex[1]) + (ex[2] + ex[3])
            live = (s16 != d16) & (ch < NCH)
            maskf = jnp.where(live, 1.0, 0.0).astype(jnp.float32)
            r = maskf / ssum
            qvecs = [ex[hh] * r for hh in range(H)]
            for k in range(16):
                qs = [_lane_bcast(qvecs[hh], k) for hh in range(H)]
                accv = [rows[b][k, pl.ds(j * 16, 16)] * qs[0]
                        for j in range(8)]
                for hh in range(1, H):
                    for j in range(8):
                        accv[j] = accv[j] + (
                            rows[b][k, pl.ds(hh * C + j * 16, 16)] * qs[hh])
                for j in range(8):
                    msg[b][k, pl.ds(j * 16, 16)] = accv[j]
            @pl.when(ch == HALF - 2)
            def _():
                pltpu.sync_copy(
                    epack_hbm.at[pl.ds(cid * E + sid * ET + ET // 2,
                                       ET // 2)], epack)
            sv[b], dv[b] = issue(ch + 2, b)
            dlane = d16 & 127
            plsc.store_scatter(msgd[b], [iota, prev_didx[b]], zerov)
            plsc.store_scatter(msgd[b], [iota, dlane], maskf)
            prev_didx[b] = dlane
            pltpu.async_copy(msg[b], acc.at[d16], sm[b], add=True)
            pltpu.async_copy(msgd[b], acc.at[NP + lax.shift_right_logical(
                d16, 7)], sm[b], add=True)
        return tuple(prev_didx), tuple(sv), tuple(dv)

    lax.fori_loop(0, NCHP // 2, body,
                  ((iota, iota), (sv0, sv1), (dv0, dv1)))
    for b in range(2):
        pltpu.make_async_copy(msg[b], acc.at[iota], sm[b]).wait()
        pltpu.make_async_copy(msgd[b], acc.at[iota], sm[b]).wait()
        pltpu.make_async_copy(h_hbm.at[iota], rows[b], sr[b]).wait()
        for hh in range(H):
            pltpu.make_async_copy(p_hbm.at[iota],
                                  psb[b].at[pl.ds(hh * 16, 16)],
                                  sp[b]).wait()
            pltpu.make_async_copy(p_hbm.at[iota],
                                  pdb[b].at[pl.ds(hh * 16, 16)],
                                  sq[b]).wait()
    plsc.subcore_barrier()

    pltpu.sync_copy(acc.at[pl.ds(NP + sid * (DROWS // NS), DROWS // NS)],
                    degbuf)

    def post(it, carry):
        roff = rbase + it * 16
        pltpu.sync_copy(acc.at[pl.ds(roff, 16)], msg0)
        dv = degbuf[lax.shift_right_logical(it, 3), pl.ds((it & 7) * 16, 16)]
        inv = 1.0 / (dv + 1.0)
        for rr in range(16):
            ir = _lane_bcast(inv, rr)
            for j in range(8):
                v = msg0[rr, pl.ds(j * 16, 16)] * ir + \
                    bias_vm[pl.ds(j * 16, 16)]
                msgd0[rr, pl.ds(j * 16, 16)] = jnp.where(v >= 0.0, v,
                                                          v * 0.2)
        pltpu.sync_copy(msgd0, out_hbm.at[cid, pl.ds(roff, 16)])
        return carry

    lax.fori_loop(0, SROWS // 16, post, jnp.int32(0))


@functools.lru_cache(maxsize=1)
def _sc_feast():
    return functools.partial(
        pl.kernel,
        out_type=jax.ShapeDtypeStruct((2, NP, 128), jnp.float32),
        mesh=plsc.VectorSubcoreMesh(core_axis_name="c",
                                    subcore_axis_name="s"),
        compiler_params=pltpu.CompilerParams(needs_layout_passes=False),
        scratch_types=[
            pltpu.VMEM((ET // 2,), jnp.int32),
            pltpu.VMEM((16, HC), jnp.float32),
            pltpu.VMEM((16, HC), jnp.float32),
            pltpu.VMEM((64,), jnp.float32),
            pltpu.VMEM((64,), jnp.float32),
            pltpu.VMEM((64,), jnp.float32),
            pltpu.VMEM((64,), jnp.float32),
            pltpu.VMEM((16, C), jnp.float32),
            pltpu.VMEM((16, C), jnp.float32),
            pltpu.VMEM((16, C), jnp.float32),
            pltpu.VMEM((16, C), jnp.float32),
            pltpu.VMEM_SHARED((NP + DROWS, C), jnp.float32),
            pltpu.VMEM((DROWS // NS, C), jnp.float32),
            pltpu.VMEM((144,), jnp.float32),
            pltpu.SemaphoreType.DMA,
            pltpu.SemaphoreType.DMA,
            pltpu.SemaphoreType.DMA,
            pltpu.SemaphoreType.DMA,
            pltpu.SemaphoreType.DMA,
            pltpu.SemaphoreType.DMA,
            pltpu.SemaphoreType.DMA,
            pltpu.SemaphoreType.DMA,
        ],
    )(_sc_body)


def kernel(x_v, x_f, wv_weight, wv_u, wv_c, wv_bias,
           wf_weight, wf_u, wf_c, wf_bias, edge_index_v, edge_index_f):
    x_all = jnp.stack([x_v, x_f])
    x_all = jnp.pad(x_all, ((0, 0), (0, NP - N), (0, 0)))
    w_all = jnp.stack([wv_weight, wf_weight])
    u_all = jnp.stack([wv_u, wf_u])
    c_all = jnp.stack([wv_c, wf_c]).reshape(2, 1, H)
    bias_all = jnp.concatenate(
        [jnp.stack([wv_bias, wf_bias]),
         jnp.stack([wv_c, wf_c]),
         jnp.zeros((2, 12), jnp.float32)], axis=1)
    src_all = jnp.concatenate([edge_index_v[0], edge_index_f[0]])
    dst_all = jnp.concatenate([edge_index_v[1], edge_index_f[1]])
    epack_all = src_all | (dst_all << 14)
    h_all, p_all, lmx_all = _pre(x_all, w_all, u_all, c_all)
    p_flat = p_all.reshape(2 * NP * H)
    out = _sc_feast()(h_all, p_flat, lmx_all, epack_all, bias_all)
    return out[0, :N], out[1, :N]
